# Initial kernel scaffold; baseline (speedup 1.0000x reference)
import jax
import jax.numpy as jnp
from jax.experimental import pallas as pl


def kernel(X, Edge_index, Edge_weight, Batching, W1, b1, W2, b2, PW1, Pb1, PW2, Pb2, PW3, Pb3, TW1, Tb1, TW2, Tb2, TW3, Tb3):
    raise NotImplementedError("write your pallas kernel here")


# speedup vs baseline: 47.1374x; 47.1374x over previous
import functools

import jax
import jax.numpy as jnp
from jax import lax
from jax.experimental import pallas as pl
from jax.experimental.pallas import tpu as pltpu
from jax.experimental.pallas import tpu_sc as plsc

N = 100000
E = 3200000
IN_CH = 128
F = 16
NGR = 64

NC = 2
NS = 16
NW = NC * NS
L = 128
GROUPS = E // L
CH = 8
NCHUNKS = GROUPS // CH
CPW = -(-NCHUNKS // NW)
NPAD = 100096
RPS = NPAD // NS
ZR = RPS // 16

_mesh = plsc.VectorSubcoreMesh(core_axis_name="c", subcore_axis_name="s")


def _zero_acc_rows(zbuf, acc, sid):
    def zfill(i, c):
        zbuf[i, :] = jnp.zeros((16,), jnp.float32)
        return c
    lax.fori_loop(0, ZR, zfill, 0)
    for k in range(RPS // ZR):
        pltpu.sync_copy(zbuf, acc.at[pl.ds(sid * RPS + k * ZR, ZR)])


def _zero_acc_1d(zbuf1, acc1, sid):
    def zfill(i, c):
        zbuf1[pl.ds(i * 16, 16)] = jnp.zeros((16,), jnp.float32)
        return c
    lax.fori_loop(0, RPS // 16, zfill, 0)
    pltpu.sync_copy(zbuf1, acc1.at[pl.ds(sid * RPS, RPS)])


@functools.partial(
    pl.kernel,
    out_type=jax.ShapeDtypeStruct((NC * NPAD,), jnp.float32),
    mesh=_mesh,
    scratch_types=[
        pltpu.VMEM((CH, L), jnp.int32),
        pltpu.VMEM((CH, L), jnp.float32),
        pltpu.VMEM((RPS,), jnp.float32),
        pltpu.VMEM_SHARED((NPAD,), jnp.float32),
        pltpu.SemaphoreType.DMA,
    ],
    compiler_params=pltpu.CompilerParams(use_tc_tiling_on_sc=False),
)
def _deg_kernel(col2d, ew2d, out, colv, eww, zbuf1, acc1, ssem):
    cid = lax.axis_index("c")
    sid = lax.axis_index("s")
    wid = sid * NC + cid

    _zero_acc_1d(zbuf1, acc1, sid)
    plsc.subcore_barrier()

    def do_groups(grow):
        pltpu.sync_copy(col2d.at[pl.ds(grow, CH)], colv)
        pltpu.sync_copy(ew2d.at[pl.ds(grow, CH)], eww)
        ds = [pltpu.async_copy(eww.at[j], acc1.at[colv.at[j]], ssem, add=True)
              for j in range(CH)]
        for d in ds:
            d.wait()

    def chunk(k, c):
        cidx = wid + k * NW

        @pl.when(cidx < NCHUNKS)
        def _():
            do_groups(cidx * CH)
        return c
    lax.fori_loop(0, CPW, chunk, 0)

    plsc.subcore_barrier()
    pltpu.sync_copy(acc1.at[pl.ds(sid * RPS, RPS)], zbuf1)
    pltpu.sync_copy(zbuf1, out.at[pl.ds(cid * NPAD + sid * RPS, RPS)])


@functools.partial(
    pl.kernel,
    out_type=jax.ShapeDtypeStruct((NC, NPAD, F), jnp.float32),
    mesh=_mesh,
    scratch_types=[
        pltpu.VMEM((CH, L), jnp.int32),
        pltpu.VMEM((CH, L), jnp.int32),
        pltpu.VMEM((CH, L), jnp.float32),
        pltpu.VMEM((CH * L, F), jnp.float32),
        pltpu.VMEM((ZR, F), jnp.float32),
        pltpu.VMEM_SHARED((NPAD, F), jnp.float32),
        pltpu.SemaphoreType.DMA,
        pltpu.SemaphoreType.DMA,
    ],
    compiler_params=pltpu.CompilerParams(use_tc_tiling_on_sc=False),
)
def _msg_kernel(row2d, col2d, ew2d, y, out, roww, colv, eww, rows, zbuf,
                acc, gsem, ssem):
    cid = lax.axis_index("c")
    sid = lax.axis_index("s")
    wid = sid * NC + cid

    _zero_acc_rows(zbuf, acc, sid)
    plsc.subcore_barrier()

    def do_groups(grow):
        pltpu.sync_copy(row2d.at[pl.ds(grow, CH)], roww)
        pltpu.sync_copy(col2d.at[pl.ds(grow, CH)], colv)
        pltpu.sync_copy(ew2d.at[pl.ds(grow, CH)], eww)
        ds = [pltpu.async_copy(y.at[roww.at[j]], rows.at[pl.ds(j * L, L)], gsem)
              for j in range(CH)]
        for d in ds:
            d.wait()

        def scale(i, c):
            g = i >> 3
            k16 = i & 7
            wv = eww[g, pl.ds(k16 * 16, 16)]
            ebase = i * 16
            for k in range(16):
                rows[ebase + k, :] = rows[ebase + k, :] * wv[k]
            return c
        lax.fori_loop(0, CH * (L // 16), scale, 0)

        ds2 = [pltpu.async_copy(rows.at[pl.ds(j * L, L)], acc.at[colv.at[j]],
                                ssem, add=True) for j in range(CH)]
        for d in ds2:
            d.wait()

    def chunk(k, c):
        cidx = wid + k * NW

        @pl.when(cidx < NCHUNKS)
        def _():
            do_groups(cidx * CH)
        return c
    lax.fori_loop(0, CPW, chunk, 0)

    plsc.subcore_barrier()
    for k in range(RPS // ZR):
        pltpu.sync_copy(acc.at[pl.ds(sid * RPS + k * ZR, ZR)], zbuf)
        pltpu.sync_copy(zbuf, out.at[cid, pl.ds(sid * RPS + k * ZR, ZR)])


_BB = 2000
_NB = N // _BB


def _leaky(x):
    return jnp.where(x > 0, x, 0.01 * x)


def _tc_a_body(x_ref, w_ref, deg_ref, dis_ref, y_ref):
    deg = deg_ref[0, :, 0] + deg_ref[1, :, 0] + 1.0
    dis = jnp.where(deg > 0, lax.rsqrt(deg), 0.0).reshape(_BB, 1)
    xw = jnp.dot(x_ref[...], w_ref[...], preferred_element_type=jnp.float32)
    dis_ref[...] = dis
    y_ref[...] = dis * xw


def _tc_b_body(s_ref, y1_ref, dis_ref, w2_ref, b1_ref, y2_ref):
    dis = dis_ref[...]
    t = dis * (s_ref[0] + s_ref[1] + y1_ref[...]) + b1_ref[...]
    o1 = _leaky(t)
    y2_ref[...] = dis * jnp.dot(o1, w2_ref[...],
                                preferred_element_type=jnp.float32)


def _tc_c_body(s_ref, y2_ref, dis_ref, b2_ref, bat_ref,
               pw1, pb1, pw2, pb2, pw3, pb3,
               tw1, tb1, tw2, tb2, tw3, tb3,
               out_ref, pooled_acc, cnt_acc):
    i = pl.program_id(0)

    @pl.when(i == 0)
    def _():
        pooled_acc[...] = jnp.zeros_like(pooled_acc)
        cnt_acc[...] = jnp.zeros_like(cnt_acc)

    dis = dis_ref[...]
    t = dis * (s_ref[0] + s_ref[1] + y2_ref[...]) + b2_ref[...]
    o2 = _leaky(t)

    gids = lax.broadcasted_iota(jnp.int32, (1, NGR), 1)
    m = (bat_ref[...] == gids).astype(jnp.float32)
    pooled_acc[...] += lax.dot_general(
        m, o2, (((0,), (0,)), ((), ())), preferred_element_type=jnp.float32)
    cnt_acc[...] += lax.dot_general(
        m, jnp.ones((_BB, 1), jnp.float32), (((0,), (0,)), ((), ())),
        preferred_element_type=jnp.float32)

    @pl.when(i == _NB - 1)
    def _():
        pm = pooled_acc[...] / jnp.maximum(cnt_acc[...], 1.0)
        phi = _leaky(jnp.dot(pm, pw1[...], preferred_element_type=jnp.float32) + pb1[...])
        phi = _leaky(jnp.dot(phi, pw2[...], preferred_element_type=jnp.float32) + pb2[...])
        phi = jnp.dot(phi, pw3[...], preferred_element_type=jnp.float32) + pb3[...]
        th = _leaky(jnp.dot(pm, tw1[...], preferred_element_type=jnp.float32) + tb1[...])
        th = _leaky(jnp.dot(th, tw2[...], preferred_element_type=jnp.float32) + tb2[...])
        th = jnp.dot(th, tw3[...], preferred_element_type=jnp.float32) + tb3[...]
        out_ref[...] = jnp.concatenate((phi, th), axis=1)


def _tc_a(X, W1, deg2):
    return pl.pallas_call(
        _tc_a_body,
        grid=(_NB,),
        in_specs=[
            pl.BlockSpec((_BB, IN_CH), lambda i: (i, 0)),
            pl.BlockSpec((IN_CH, F), lambda i: (0, 0)),
            pl.BlockSpec((NC, _BB, 1), lambda i: (0, i, 0)),
        ],
        out_specs=[
            pl.BlockSpec((_BB, 1), lambda i: (i, 0)),
            pl.BlockSpec((_BB, F), lambda i: (i, 0)),
        ],
        out_shape=[
            jax.ShapeDtypeStruct((N, 1), jnp.float32),
            jax.ShapeDtypeStruct((N, F), jnp.float32),
        ],
    )(X, W1, deg2)


def _tc_b(S1, y1, dis, W2, b1):
    return pl.pallas_call(
        _tc_b_body,
        grid=(_NB,),
        in_specs=[
            pl.BlockSpec((NC, _BB, F), lambda i: (0, i, 0)),
            pl.BlockSpec((_BB, F), lambda i: (i, 0)),
            pl.BlockSpec((_BB, 1), lambda i: (i, 0)),
            pl.BlockSpec((F, F), lambda i: (0, 0)),
            pl.BlockSpec((1, F), lambda i: (0, 0)),
        ],
        out_specs=pl.BlockSpec((_BB, F), lambda i: (i, 0)),
        out_shape=jax.ShapeDtypeStruct((N, F), jnp.float32),
    )(S1, y1, dis, W2, b1)


def _tc_c(S2, y2, dis, b2, bat, mlp):
    small = lambda r, c: pl.BlockSpec((r, c), lambda i: (0, 0))
    return pl.pallas_call(
        _tc_c_body,
        grid=(_NB,),
        in_specs=[
            pl.BlockSpec((NC, _BB, F), lambda i: (0, i, 0)),
            pl.BlockSpec((_BB, F), lambda i: (i, 0)),
            pl.BlockSpec((_BB, 1), lambda i: (i, 0)),
            pl.BlockSpec((1, F), lambda i: (0, 0)),
            pl.BlockSpec((_BB, 1), lambda i: (i, 0)),
            small(F, F), small(1, F), small(F, F), small(1, F),
            small(F, 2), small(1, 2),
            small(F, F), small(1, F), small(F, F), small(1, F),
            small(F, 2), small(1, 2),
        ],
        out_specs=pl.BlockSpec((NGR, 4), lambda i: (0, 0)),
        out_shape=jax.ShapeDtypeStruct((NGR, 4), jnp.float32),
        scratch_shapes=[
            pltpu.VMEM((NGR, F), jnp.float32),
            pltpu.VMEM((NGR, 1), jnp.float32),
        ],
    )(S2, y2, dis, b2, bat, *mlp)


def kernel(X, Edge_index, Edge_weight, Batching, W1, b1, W2, b2,
           PW1, Pb1, PW2, Pb2, PW3, Pb3, TW1, Tb1, TW2, Tb2, TW3, Tb3):
    row2d = Edge_index[0].astype(jnp.int32).reshape(GROUPS, L)
    col2d = Edge_index[1].astype(jnp.int32).reshape(GROUPS, L)
    ew2d = Edge_weight.reshape(GROUPS, L)
    bat = Batching.astype(jnp.int32).reshape(N, 1)

    deg2 = _deg_kernel(col2d, ew2d)
    dis, y1 = _tc_a(X, W1, deg2.reshape(NC, NPAD, 1))
    S1 = _msg_kernel(row2d, col2d, ew2d, y1)
    y2 = _tc_b(S1, y1, dis, W2, b1.reshape(1, F))
    S2 = _msg_kernel(row2d, col2d, ew2d, y2)
    mlp = (PW1, Pb1.reshape(1, F), PW2, Pb2.reshape(1, F), PW3, Pb3.reshape(1, 2),
           TW1, Tb1.reshape(1, F), TW2, Tb2.reshape(1, F), TW3, Tb3.reshape(1, 2))
    return _tc_c(S2, y2, dis, b2.reshape(1, F), bat, mlp)


# speedup vs baseline: 63.0106x; 1.3367x over previous
import functools

import jax
import jax.numpy as jnp
from jax import lax
from jax.experimental import pallas as pl
from jax.experimental.pallas import tpu as pltpu
from jax.experimental.pallas import tpu_sc as plsc

N = 100000
E = 3200000
IN_CH = 128
F = 16
NGR = 64

NC = 2
NS = 16
NW = NC * NS
L = 128
GROUPS = E // L
CH = 4
NCHUNKS = GROUPS // CH
CPW = -(-NCHUNKS // NW)
PAIRS = CPW // 2
NPAD = 100096
RPS = NPAD // NS
ZR = RPS // 16

_mesh = plsc.VectorSubcoreMesh(core_axis_name="c", subcore_axis_name="s")


def _zero_acc_rows(zbuf, acc, sid):
    def zfill(i, c):
        zbuf[i, :] = jnp.zeros((16,), jnp.float32)
        return c
    lax.fori_loop(0, ZR, zfill, 0)
    for k in range(RPS // ZR):
        pltpu.sync_copy(zbuf.at[pl.ds(0, ZR)],
                        acc.at[pl.ds(sid * RPS + k * ZR, ZR)])


def _zero_acc_1d(zbuf1, acc1, sid):
    def zfill(i, c):
        zbuf1[pl.ds(i * 16, 16)] = jnp.zeros((16,), jnp.float32)
        return c
    lax.fori_loop(0, RPS // 16, zfill, 0)
    pltpu.sync_copy(zbuf1, acc1.at[pl.ds(sid * RPS, RPS)])


def _zero_ew(eww):
    def zf(i, c):
        g = i >> 3
        k16 = i & 7
        eww[g, pl.ds(k16 * 16, 16)] = jnp.zeros((16,), jnp.float32)
        return c
    lax.fori_loop(0, CH * (L // 16), zf, 0)


def _chunk_of(wid, slot):
    cidx = wid + slot * NW
    return jnp.minimum(cidx, NCHUNKS - 1), cidx >= NCHUNKS


@functools.partial(
    pl.kernel,
    out_type=jax.ShapeDtypeStruct((NC * NPAD,), jnp.float32),
    mesh=_mesh,
    scratch_types=[
        [pltpu.VMEM((CH, L), jnp.int32) for _ in range(2)],
        [pltpu.VMEM((CH, L), jnp.float32) for _ in range(2)],
        pltpu.VMEM((RPS,), jnp.float32),
        pltpu.VMEM_SHARED((NPAD,), jnp.float32),
        pltpu.SemaphoreType.DMA,
        pltpu.SemaphoreType.DMA,
    ],
    compiler_params=pltpu.CompilerParams(use_tc_tiling_on_sc=False),
)
def _deg_kernel(col2d, ew2d, out, colv, eww, zbuf1, acc1, lsem, ssem):
    cid = lax.axis_index("c")
    sid = lax.axis_index("s")
    wid = sid * NC + cid

    _zero_acc_1d(zbuf1, acc1, sid)
    plsc.subcore_barrier()

    def load(cidx, b, oob):
        l0 = pltpu.async_copy(col2d.at[pl.ds(cidx * CH, CH)], colv[b], lsem)
        l1 = pltpu.async_copy(ew2d.at[pl.ds(cidx * CH, CH)], eww[b], lsem)
        l0.wait()
        l1.wait()

        @pl.when(oob)
        def _():
            _zero_ew(eww[b])

    c0, o0 = _chunk_of(wid, 0)
    load(c0, 0, o0)

    def pair(k2, c):
        for h in range(2):
            nxt_c, nxt_oob = _chunk_of(wid, 2 * k2 + h + 1)
            ds = [pltpu.async_copy(eww[h].at[j], acc1.at[colv[h].at[j]],
                                   ssem, add=True) for j in range(CH)]
            load(nxt_c, 1 - h, nxt_oob)
            for d in ds:
                d.wait()
        return c
    lax.fori_loop(0, PAIRS, pair, 0)

    plsc.subcore_barrier()
    pltpu.sync_copy(acc1.at[pl.ds(sid * RPS, RPS)], zbuf1)
    pltpu.sync_copy(zbuf1, out.at[pl.ds(cid * NPAD + sid * RPS, RPS)])


@functools.partial(
    pl.kernel,
    out_type=jax.ShapeDtypeStruct((NC, NPAD, F), jnp.float32),
    mesh=_mesh,
    scratch_types=[
        [pltpu.VMEM((CH, L), jnp.int32) for _ in range(2)],
        [pltpu.VMEM((CH, L), jnp.int32) for _ in range(2)],
        [pltpu.VMEM((CH, L), jnp.float32) for _ in range(2)],
        [pltpu.VMEM((CH * L, F), jnp.float32) for _ in range(2)],
        pltpu.VMEM_SHARED((NPAD, F), jnp.float32),
        pltpu.SemaphoreType.DMA,
        [pltpu.SemaphoreType.DMA for _ in range(2)],
        pltpu.SemaphoreType.DMA,
    ],
    compiler_params=pltpu.CompilerParams(use_tc_tiling_on_sc=False),
)
def _msg_kernel(row2d, col2d, ew2d, y, out, roww, colv, eww, rows,
                acc, lsem, gsem, ssem):
    cid = lax.axis_index("c")
    sid = lax.axis_index("s")
    wid = sid * NC + cid

    zview = rows[0].at[pl.ds(0, ZR)]
    _zero_acc_rows(rows[0], acc, sid)
    plsc.subcore_barrier()

    def fire_gathers(b):
        return [pltpu.async_copy(y.at[roww[b].at[j]],
                                 rows[b].at[pl.ds(j * L, L)], gsem[b])
                for j in range(CH)]

    def drain_gathers(b):
        for j in range(CH):
            pltpu.make_async_copy(y.at[roww[b].at[j]],
                                  rows[b].at[pl.ds(j * L, L)], gsem[b]).wait()

    def load(cidx, b, oob):
        ls = [pltpu.async_copy(row2d.at[pl.ds(cidx * CH, CH)], roww[b], lsem),
              pltpu.async_copy(col2d.at[pl.ds(cidx * CH, CH)], colv[b], lsem),
              pltpu.async_copy(ew2d.at[pl.ds(cidx * CH, CH)], eww[b], lsem)]
        for d in ls:
            d.wait()

        @pl.when(oob)
        def _():
            _zero_ew(eww[b])

    def scale(b):
        def body(i, c):
            g = i >> 3
            k16 = i & 7
            wv = eww[b][g, pl.ds(k16 * 16, 16)]
            ebase = i * 16
            for k in range(16):
                rows[b][ebase + k, :] = rows[b][ebase + k, :] * wv[k]
            return c
        lax.fori_loop(0, CH * (L // 16), body, 0)

    c0, o0 = _chunk_of(wid, 0)
    load(c0, 0, o0)
    fire_gathers(0)

    def pair(k2, c):
        for h in range(2):
            nxt_c, nxt_oob = _chunk_of(wid, 2 * k2 + h + 1)
            load(nxt_c, 1 - h, nxt_oob)
            drain_gathers(h)
            fire_gathers(1 - h)
            scale(h)
            ds = [pltpu.async_copy(rows[h].at[pl.ds(j * L, L)],
                                   acc.at[colv[h].at[j]], ssem, add=True)
                  for j in range(CH)]
            for d in ds:
                d.wait()
        return c
    lax.fori_loop(0, PAIRS, pair, 0)

    drain_gathers(0)

    plsc.subcore_barrier()
    for k in range(RPS // ZR):
        pltpu.sync_copy(acc.at[pl.ds(sid * RPS + k * ZR, ZR)], zview)
        pltpu.sync_copy(zview, out.at[cid, pl.ds(sid * RPS + k * ZR, ZR)])


_BB = 2000
_NB = N // _BB


def _leaky(x):
    return jnp.where(x > 0, x, 0.01 * x)


def _tc_a_body(x_ref, w_ref, deg_ref, dis_ref, y_ref):
    deg = deg_ref[0, :, 0] + deg_ref[1, :, 0] + 1.0
    dis = jnp.where(deg > 0, lax.rsqrt(deg), 0.0).reshape(_BB, 1)
    xw = jnp.dot(x_ref[...], w_ref[...], preferred_element_type=jnp.float32)
    dis_ref[...] = dis
    y_ref[...] = dis * xw


def _tc_b_body(s_ref, y1_ref, dis_ref, w2_ref, b1_ref, y2_ref):
    dis = dis_ref[...]
    t = dis * (s_ref[0] + s_ref[1] + y1_ref[...]) + b1_ref[...]
    o1 = _leaky(t)
    y2_ref[...] = dis * jnp.dot(o1, w2_ref[...],
                                preferred_element_type=jnp.float32)


def _tc_c_body(s_ref, y2_ref, dis_ref, b2_ref, bat_ref,
               pw1, pb1, pw2, pb2, pw3, pb3,
               tw1, tb1, tw2, tb2, tw3, tb3,
               out_ref, pooled_acc, cnt_acc):
    i = pl.program_id(0)

    @pl.when(i == 0)
    def _():
        pooled_acc[...] = jnp.zeros_like(pooled_acc)
        cnt_acc[...] = jnp.zeros_like(cnt_acc)

    dis = dis_ref[...]
    t = dis * (s_ref[0] + s_ref[1] + y2_ref[...]) + b2_ref[...]
    o2 = _leaky(t)

    gids = lax.broadcasted_iota(jnp.int32, (1, NGR), 1)
    m = (bat_ref[...] == gids).astype(jnp.float32)
    pooled_acc[...] += lax.dot_general(
        m, o2, (((0,), (0,)), ((), ())), preferred_element_type=jnp.float32)
    cnt_acc[...] += lax.dot_general(
        m, jnp.ones((_BB, 1), jnp.float32), (((0,), (0,)), ((), ())),
        preferred_element_type=jnp.float32)

    @pl.when(i == _NB - 1)
    def _():
        pm = pooled_acc[...] / jnp.maximum(cnt_acc[...], 1.0)
        phi = _leaky(jnp.dot(pm, pw1[...], preferred_element_type=jnp.float32) + pb1[...])
        phi = _leaky(jnp.dot(phi, pw2[...], preferred_element_type=jnp.float32) + pb2[...])
        phi = jnp.dot(phi, pw3[...], preferred_element_type=jnp.float32) + pb3[...]
        th = _leaky(jnp.dot(pm, tw1[...], preferred_element_type=jnp.float32) + tb1[...])
        th = _leaky(jnp.dot(th, tw2[...], preferred_element_type=jnp.float32) + tb2[...])
        th = jnp.dot(th, tw3[...], preferred_element_type=jnp.float32) + tb3[...]
        out_ref[...] = jnp.concatenate((phi, th), axis=1)


def _tc_a(X, W1, deg2):
    return pl.pallas_call(
        _tc_a_body,
        grid=(_NB,),
        in_specs=[
            pl.BlockSpec((_BB, IN_CH), lambda i: (i, 0)),
            pl.BlockSpec((IN_CH, F), lambda i: (0, 0)),
            pl.BlockSpec((NC, _BB, 1), lambda i: (0, i, 0)),
        ],
        out_specs=[
            pl.BlockSpec((_BB, 1), lambda i: (i, 0)),
            pl.BlockSpec((_BB, F), lambda i: (i, 0)),
        ],
        out_shape=[
            jax.ShapeDtypeStruct((N, 1), jnp.float32),
            jax.ShapeDtypeStruct((N, F), jnp.float32),
        ],
    )(X, W1, deg2)


def _tc_b(S1, y1, dis, W2, b1):
    return pl.pallas_call(
        _tc_b_body,
        grid=(_NB,),
        in_specs=[
            pl.BlockSpec((NC, _BB, F), lambda i: (0, i, 0)),
            pl.BlockSpec((_BB, F), lambda i: (i, 0)),
            pl.BlockSpec((_BB, 1), lambda i: (i, 0)),
            pl.BlockSpec((F, F), lambda i: (0, 0)),
            pl.BlockSpec((1, F), lambda i: (0, 0)),
        ],
        out_specs=pl.BlockSpec((_BB, F), lambda i: (i, 0)),
        out_shape=jax.ShapeDtypeStruct((N, F), jnp.float32),
    )(S1, y1, dis, W2, b1)


def _tc_c(S2, y2, dis, b2, bat, mlp):
    small = lambda r, c: pl.BlockSpec((r, c), lambda i: (0, 0))
    return pl.pallas_call(
        _tc_c_body,
        grid=(_NB,),
        in_specs=[
            pl.BlockSpec((NC, _BB, F), lambda i: (0, i, 0)),
            pl.BlockSpec((_BB, F), lambda i: (i, 0)),
            pl.BlockSpec((_BB, 1), lambda i: (i, 0)),
            pl.BlockSpec((1, F), lambda i: (0, 0)),
            pl.BlockSpec((_BB, 1), lambda i: (i, 0)),
            small(F, F), small(1, F), small(F, F), small(1, F),
            small(F, 2), small(1, 2),
            small(F, F), small(1, F), small(F, F), small(1, F),
            small(F, 2), small(1, 2),
        ],
        out_specs=pl.BlockSpec((NGR, 4), lambda i: (0, 0)),
        out_shape=jax.ShapeDtypeStruct((NGR, 4), jnp.float32),
        scratch_shapes=[
            pltpu.VMEM((NGR, F), jnp.float32),
            pltpu.VMEM((NGR, 1), jnp.float32),
        ],
    )(S2, y2, dis, b2, bat, *mlp)


def kernel(X, Edge_index, Edge_weight, Batching, W1, b1, W2, b2,
           PW1, Pb1, PW2, Pb2, PW3, Pb3, TW1, Tb1, TW2, Tb2, TW3, Tb3):
    row2d = Edge_index[0].astype(jnp.int32).reshape(GROUPS, L)
    col2d = Edge_index[1].astype(jnp.int32).reshape(GROUPS, L)
    ew2d = Edge_weight.reshape(GROUPS, L)
    bat = Batching.astype(jnp.int32).reshape(N, 1)

    deg2 = _deg_kernel(col2d, ew2d)
    dis, y1 = _tc_a(X, W1, deg2.reshape(NC, NPAD, 1))
    S1 = _msg_kernel(row2d, col2d, ew2d, y1)
    y2 = _tc_b(S1, y1, dis, W2, b1.reshape(1, F))
    S2 = _msg_kernel(row2d, col2d, ew2d, y2)
    mlp = (PW1, Pb1.reshape(1, F), PW2, Pb2.reshape(1, F), PW3, Pb3.reshape(1, 2),
           TW1, Tb1.reshape(1, F), TW2, Tb2.reshape(1, F), TW3, Tb3.reshape(1, 2))
    return _tc_c(S2, y2, dis, b2.reshape(1, F), bat, mlp)


# speedup vs baseline: 66.9077x; 1.0618x over previous
import functools

import jax
import jax.numpy as jnp
from jax import lax
from jax.experimental import pallas as pl
from jax.experimental.pallas import tpu as pltpu
from jax.experimental.pallas import tpu_sc as plsc

N = 100000
E = 3200000
IN_CH = 128
F = 16
NGR = 64

NC = 2
NS = 16
NW = NC * NS
L = 128
GROUPS = E // L
CH = 4
NCHUNKS = GROUPS // CH
CPW = -(-NCHUNKS // NW)
PAIRS = CPW // 2
NPAD = 100096
RPS = NPAD // NS
ZR = RPS // 16

_mesh = plsc.VectorSubcoreMesh(core_axis_name="c", subcore_axis_name="s")


def _zero_acc_rows(zbuf, acc, sid):
    def zfill(i, c):
        zbuf[i, :] = jnp.zeros((16,), jnp.float32)
        return c
    lax.fori_loop(0, ZR, zfill, 0)
    for k in range(RPS // ZR):
        pltpu.sync_copy(zbuf.at[pl.ds(0, ZR)],
                        acc.at[pl.ds(sid * RPS + k * ZR, ZR)])


def _zero_acc_1d(zbuf1, acc1, sid):
    def zfill(i, c):
        zbuf1[pl.ds(i * 16, 16)] = jnp.zeros((16,), jnp.float32)
        return c
    lax.fori_loop(0, RPS // 16, zfill, 0)
    pltpu.sync_copy(zbuf1, acc1.at[pl.ds(sid * RPS, RPS)])


def _zero_ew(eww):
    def zf(i, c):
        g = i >> 3
        k16 = i & 7
        eww[g, pl.ds(k16 * 16, 16)] = jnp.zeros((16,), jnp.float32)
        return c
    lax.fori_loop(0, CH * (L // 16), zf, 0)


def _chunk_of(wid, slot):
    cidx = wid + slot * NW
    return jnp.minimum(cidx, NCHUNKS - 1), cidx >= NCHUNKS


@functools.partial(
    pl.kernel,
    out_type=jax.ShapeDtypeStruct((NC * NPAD,), jnp.float32),
    mesh=_mesh,
    scratch_types=[
        [pltpu.VMEM((CH, L), jnp.int32) for _ in range(2)],
        [pltpu.VMEM((CH, L), jnp.float32) for _ in range(2)],
        pltpu.VMEM((RPS,), jnp.float32),
        pltpu.VMEM_SHARED((NPAD,), jnp.float32),
        pltpu.SemaphoreType.DMA,
        pltpu.SemaphoreType.DMA,
    ],
    compiler_params=pltpu.CompilerParams(use_tc_tiling_on_sc=False),
)
def _deg_kernel(col2d, ew2d, out, colv, eww, zbuf1, acc1, lsem, ssem):
    cid = lax.axis_index("c")
    sid = lax.axis_index("s")
    wid = sid * NC + cid

    _zero_acc_1d(zbuf1, acc1, sid)
    plsc.subcore_barrier()

    def load(cidx, b, oob):
        l0 = pltpu.async_copy(col2d.at[pl.ds(cidx * CH, CH)], colv[b], lsem)
        l1 = pltpu.async_copy(ew2d.at[pl.ds(cidx * CH, CH)], eww[b], lsem)
        l0.wait()
        l1.wait()

        @pl.when(oob)
        def _():
            _zero_ew(eww[b])

    c0, o0 = _chunk_of(wid, 0)
    load(c0, 0, o0)

    def pair(k2, c):
        for h in range(2):
            nxt_c, nxt_oob = _chunk_of(wid, 2 * k2 + h + 1)
            ds = [pltpu.async_copy(eww[h].at[j], acc1.at[colv[h].at[j]],
                                   ssem, add=True) for j in range(CH)]
            load(nxt_c, 1 - h, nxt_oob)
            for d in ds:
                d.wait()
        return c
    lax.fori_loop(0, PAIRS, pair, 0)

    plsc.subcore_barrier()
    pltpu.sync_copy(acc1.at[pl.ds(sid * RPS, RPS)], zbuf1)
    pltpu.sync_copy(zbuf1, out.at[pl.ds(cid * NPAD + sid * RPS, RPS)])


@functools.partial(
    pl.kernel,
    out_type=jax.ShapeDtypeStruct((NC, NPAD, F), jnp.float32),
    mesh=_mesh,
    scratch_types=[
        [pltpu.VMEM((CH, L), jnp.int32) for _ in range(2)],
        [pltpu.VMEM((CH, L), jnp.int32) for _ in range(2)],
        [pltpu.VMEM((CH, L), jnp.float32) for _ in range(2)],
        [pltpu.VMEM((CH * L, F), jnp.float32) for _ in range(2)],
        pltpu.VMEM_SHARED((NPAD, F), jnp.float32),
        pltpu.SemaphoreType.DMA,
        [pltpu.SemaphoreType.DMA for _ in range(2)],
        pltpu.SemaphoreType.DMA,
    ],
    compiler_params=pltpu.CompilerParams(use_tc_tiling_on_sc=False),
)
def _msg_kernel(row2d, col2d, ew2d, y, out, roww, colv, eww, rows,
                acc, lsem, gsem, ssem):
    cid = lax.axis_index("c")
    sid = lax.axis_index("s")
    wid = sid * NC + cid

    zview = rows[0].at[pl.ds(0, ZR)]
    _zero_acc_rows(rows[0], acc, sid)
    plsc.subcore_barrier()

    def fire_gathers(b):
        return [pltpu.async_copy(y.at[roww[b].at[j]],
                                 rows[b].at[pl.ds(j * L, L)], gsem[b])
                for j in range(CH)]

    def drain_gathers(b):
        for j in range(CH):
            pltpu.make_async_copy(y.at[roww[b].at[j]],
                                  rows[b].at[pl.ds(j * L, L)], gsem[b]).wait()

    def load(cidx, b, oob):
        ls = [pltpu.async_copy(row2d.at[pl.ds(cidx * CH, CH)], roww[b], lsem),
              pltpu.async_copy(col2d.at[pl.ds(cidx * CH, CH)], colv[b], lsem),
              pltpu.async_copy(ew2d.at[pl.ds(cidx * CH, CH)], eww[b], lsem)]
        for d in ls:
            d.wait()

        @pl.when(oob)
        def _():
            _zero_ew(eww[b])

    def scale(b):
        @functools.partial(plsc.parallel_loop, 0, CH * (L // 16), unroll=2)
        def _(i):
            g = i >> 3
            k16 = i & 7
            wv = eww[b][g, pl.ds(k16 * 16, 16)]
            ebase = i * 16
            for k in range(16):
                rows[b][ebase + k, :] = rows[b][ebase + k, :] * wv[k]

    c0, o0 = _chunk_of(wid, 0)
    load(c0, 0, o0)
    fire_gathers(0)

    def pair(k2, c):
        for h in range(2):
            nxt_c, nxt_oob = _chunk_of(wid, 2 * k2 + h + 1)
            load(nxt_c, 1 - h, nxt_oob)
            drain_gathers(h)
            fire_gathers(1 - h)
            scale(h)
            ds = [pltpu.async_copy(rows[h].at[pl.ds(j * L, L)],
                                   acc.at[colv[h].at[j]], ssem, add=True)
                  for j in range(CH)]
            for d in ds:
                d.wait()
        return c
    lax.fori_loop(0, PAIRS, pair, 0)

    drain_gathers(0)

    plsc.subcore_barrier()
    for k in range(RPS // ZR):
        pltpu.sync_copy(acc.at[pl.ds(sid * RPS + k * ZR, ZR)], zview)
        pltpu.sync_copy(zview, out.at[cid, pl.ds(sid * RPS + k * ZR, ZR)])


_BB = 2000
_NB = N // _BB


def _leaky(x):
    return jnp.where(x > 0, x, 0.01 * x)


def _tc_a_body(x_ref, w_ref, deg_ref, dis_ref, y_ref):
    deg = deg_ref[0, :, 0] + deg_ref[1, :, 0] + 1.0
    dis = jnp.where(deg > 0, lax.rsqrt(deg), 0.0).reshape(_BB, 1)
    xw = jnp.dot(x_ref[...], w_ref[...], preferred_element_type=jnp.float32)
    dis_ref[...] = dis
    y_ref[...] = dis * xw


def _tc_b_body(s_ref, y1_ref, dis_ref, w2_ref, b1_ref, y2_ref):
    dis = dis_ref[...]
    t = dis * (s_ref[0] + s_ref[1] + y1_ref[...]) + b1_ref[...]
    o1 = _leaky(t)
    y2_ref[...] = dis * jnp.dot(o1, w2_ref[...],
                                preferred_element_type=jnp.float32)


def _tc_c_body(s_ref, y2_ref, dis_ref, b2_ref, bat_ref,
               pw1, pb1, pw2, pb2, pw3, pb3,
               tw1, tb1, tw2, tb2, tw3, tb3,
               out_ref, pooled_acc, cnt_acc):
    i = pl.program_id(0)

    @pl.when(i == 0)
    def _():
        pooled_acc[...] = jnp.zeros_like(pooled_acc)
        cnt_acc[...] = jnp.zeros_like(cnt_acc)

    dis = dis_ref[...]
    t = dis * (s_ref[0] + s_ref[1] + y2_ref[...]) + b2_ref[...]
    o2 = _leaky(t)

    gids = lax.broadcasted_iota(jnp.int32, (1, NGR), 1)
    m = (bat_ref[...] == gids).astype(jnp.float32)
    pooled_acc[...] += lax.dot_general(
        m, o2, (((0,), (0,)), ((), ())), preferred_element_type=jnp.float32)
    cnt_acc[...] += lax.dot_general(
        m, jnp.ones((_BB, 1), jnp.float32), (((0,), (0,)), ((), ())),
        preferred_element_type=jnp.float32)

    @pl.when(i == _NB - 1)
    def _():
        pm = pooled_acc[...] / jnp.maximum(cnt_acc[...], 1.0)
        phi = _leaky(jnp.dot(pm, pw1[...], preferred_element_type=jnp.float32) + pb1[...])
        phi = _leaky(jnp.dot(phi, pw2[...], preferred_element_type=jnp.float32) + pb2[...])
        phi = jnp.dot(phi, pw3[...], preferred_element_type=jnp.float32) + pb3[...]
        th = _leaky(jnp.dot(pm, tw1[...], preferred_element_type=jnp.float32) + tb1[...])
        th = _leaky(jnp.dot(th, tw2[...], preferred_element_type=jnp.float32) + tb2[...])
        th = jnp.dot(th, tw3[...], preferred_element_type=jnp.float32) + tb3[...]
        out_ref[...] = jnp.concatenate((phi, th), axis=1)


def _tc_a(X, W1, deg2):
    return pl.pallas_call(
        _tc_a_body,
        grid=(_NB,),
        in_specs=[
            pl.BlockSpec((_BB, IN_CH), lambda i: (i, 0)),
            pl.BlockSpec((IN_CH, F), lambda i: (0, 0)),
            pl.BlockSpec((NC, _BB, 1), lambda i: (0, i, 0)),
        ],
        out_specs=[
            pl.BlockSpec((_BB, 1), lambda i: (i, 0)),
            pl.BlockSpec((_BB, F), lambda i: (i, 0)),
        ],
        out_shape=[
            jax.ShapeDtypeStruct((N, 1), jnp.float32),
            jax.ShapeDtypeStruct((N, F), jnp.float32),
        ],
    )(X, W1, deg2)


def _tc_b(S1, y1, dis, W2, b1):
    return pl.pallas_call(
        _tc_b_body,
        grid=(_NB,),
        in_specs=[
            pl.BlockSpec((NC, _BB, F), lambda i: (0, i, 0)),
            pl.BlockSpec((_BB, F), lambda i: (i, 0)),
            pl.BlockSpec((_BB, 1), lambda i: (i, 0)),
            pl.BlockSpec((F, F), lambda i: (0, 0)),
            pl.BlockSpec((1, F), lambda i: (0, 0)),
        ],
        out_specs=pl.BlockSpec((_BB, F), lambda i: (i, 0)),
        out_shape=jax.ShapeDtypeStruct((N, F), jnp.float32),
    )(S1, y1, dis, W2, b1)


def _tc_c(S2, y2, dis, b2, bat, mlp):
    small = lambda r, c: pl.BlockSpec((r, c), lambda i: (0, 0))
    return pl.pallas_call(
        _tc_c_body,
        grid=(_NB,),
        in_specs=[
            pl.BlockSpec((NC, _BB, F), lambda i: (0, i, 0)),
            pl.BlockSpec((_BB, F), lambda i: (i, 0)),
            pl.BlockSpec((_BB, 1), lambda i: (i, 0)),
            pl.BlockSpec((1, F), lambda i: (0, 0)),
            pl.BlockSpec((_BB, 1), lambda i: (i, 0)),
            small(F, F), small(1, F), small(F, F), small(1, F),
            small(F, 2), small(1, 2),
            small(F, F), small(1, F), small(F, F), small(1, F),
            small(F, 2), small(1, 2),
        ],
        out_specs=pl.BlockSpec((NGR, 4), lambda i: (0, 0)),
        out_shape=jax.ShapeDtypeStruct((NGR, 4), jnp.float32),
        scratch_shapes=[
            pltpu.VMEM((NGR, F), jnp.float32),
            pltpu.VMEM((NGR, 1), jnp.float32),
        ],
    )(S2, y2, dis, b2, bat, *mlp)


def kernel(X, Edge_index, Edge_weight, Batching, W1, b1, W2, b2,
           PW1, Pb1, PW2, Pb2, PW3, Pb3, TW1, Tb1, TW2, Tb2, TW3, Tb3):
    row2d = Edge_index[0].astype(jnp.int32).reshape(GROUPS, L)
    col2d = Edge_index[1].astype(jnp.int32).reshape(GROUPS, L)
    ew2d = Edge_weight.reshape(GROUPS, L)
    bat = Batching.astype(jnp.int32).reshape(N, 1)

    deg2 = _deg_kernel(col2d, ew2d)
    dis, y1 = _tc_a(X, W1, deg2.reshape(NC, NPAD, 1))
    S1 = _msg_kernel(row2d, col2d, ew2d, y1)
    y2 = _tc_b(S1, y1, dis, W2, b1.reshape(1, F))
    S2 = _msg_kernel(row2d, col2d, ew2d, y2)
    mlp = (PW1, Pb1.reshape(1, F), PW2, Pb2.reshape(1, F), PW3, Pb3.reshape(1, 2),
           TW1, Tb1.reshape(1, F), TW2, Tb2.reshape(1, F), TW3, Tb3.reshape(1, 2))
    return _tc_c(S2, y2, dis, b2.reshape(1, F), bat, mlp)
